# baseline (speedup 1.0000x reference)
import functools

import jax
import jax.numpy as jnp
from jax import lax
from jax.experimental import pallas as pl
from jax.experimental.pallas import tpu as pltpu
from jax.experimental.pallas import tpu_sc as plsc

_N = 10000
_R = 17
_E = 320000
_EMB = 16
_NODE = 123
_BINS = _R * _N
_WSL = 10624
_WSL15 = _BINS - 15 * _WSL
_CBINS = 180224
_CSL = _CBINS // 16
_KSL = _CBINS // 32
_HROWS = _N // 16
_NC = 2
_NS = 16
_NW = _NC * _NS
_CH = 128
_NCH = 80
_EPW = _NCH * _CH
_NROWS = _E // _CH
_TAIL = _NROWS - (_NW - 1) * _NCH
_TAILE = _TAIL * _CH

_mesh = plsc.VectorSubcoreMesh(core_axis_name="c", subcore_axis_name="s")
_sc_params = pltpu.CompilerParams(use_tc_tiling_on_sc=False)

_GATHER_DNUMS = lax.GatherDimensionNumbers(
    offset_dims=(), collapsed_slice_dims=(0,), start_index_map=(0,))


def _splat_lane(vec16, t):
    idx = jnp.full((16, 1), t, jnp.int32)
    return lax.gather(vec16, idx, _GATHER_DNUMS, (1,),
                      mode=lax.GatherScatterMode.PROMISE_IN_BOUNDS)


def _load_worker_edges(hbm, vmem, w):
    @pl.when(w < _NW - 1)
    def _():
        pltpu.sync_copy(hbm.at[pl.ds(w * _EPW, _EPW)], vmem)

    @pl.when(w == _NW - 1)
    def _():
        pltpu.sync_copy(hbm.at[pl.ds((_NW - 1) * _EPW, _TAILE)],
                        vmem.at[pl.ds(0, _TAILE)])


@functools.partial(
    pl.kernel,
    out_type=[
        jax.ShapeDtypeStruct((_CBINS,), jnp.float32),
        jax.ShapeDtypeStruct((_CBINS,), jnp.float32),
    ],
    mesh=_mesh,
    compiler_params=_sc_params,
    scratch_types=[
        pltpu.VMEM((_EPW,), jnp.int32),
        pltpu.VMEM((_EPW,), jnp.int32),
        pltpu.VMEM((_NCH, _CH), jnp.int32),
        pltpu.VMEM((_CH,), jnp.float32),
        pltpu.VMEM_SHARED((_CBINS,), jnp.float32),
        pltpu.SemaphoreType.DMA,
    ],
)
def _k1_counts(src_hbm, et_hbm, zeros_hbm, c0_hbm, c1_hbm,
               src_v, et_v, ver_v, ones_v, counts_sh, sem):
    c = lax.axis_index("c")
    s = lax.axis_index("s")
    w = s * _NC + c
    nch = jnp.where(w == _NW - 1, _TAIL, _NCH)
    pltpu.sync_copy(zeros_hbm.at[pl.ds(s * _CSL, _CSL)],
                    counts_sh.at[pl.ds(s * _CSL, _CSL)])
    for k in range(_CH // 16):
        ones_v[pl.ds(k * 16, 16)] = jnp.full((16,), 1.0, jnp.float32)
    _load_worker_edges(src_hbm, src_v, w)
    _load_worker_edges(et_hbm, et_v, w)

    @pl.loop(0, nch)
    def _(j):
        for k in range(_CH // 16):
            sl1 = pl.ds(j * _CH + k * 16, 16)
            ver_v[j, pl.ds(k * 16, 16)] = et_v[sl1] * _N + src_v[sl1]

    plsc.subcore_barrier()

    @pl.loop(0, nch)
    def _(j):
        pltpu.async_copy(ones_v, counts_sh.at[ver_v.at[j]], sem, add=True)

    @pl.loop(0, nch)
    def _(j):
        pltpu.make_async_copy(ones_v, counts_sh.at[ver_v.at[j]], sem).wait()

    plsc.subcore_barrier()

    @pl.when(c == 0)
    def _():
        pltpu.sync_copy(counts_sh.at[pl.ds(s * _CSL, _CSL)],
                        c0_hbm.at[pl.ds(s * _CSL, _CSL)])

    @pl.when(c == 1)
    def _():
        pltpu.sync_copy(counts_sh.at[pl.ds(s * _CSL, _CSL)],
                        c1_hbm.at[pl.ds(s * _CSL, _CSL)])


@functools.partial(
    pl.kernel,
    out_type=jax.ShapeDtypeStruct((_CBINS,), jnp.float32),
    mesh=_mesh,
    compiler_params=_sc_params,
    scratch_types=[
        pltpu.VMEM((_KSL,), jnp.float32),
        pltpu.VMEM((_KSL,), jnp.float32),
        pltpu.VMEM((_KSL,), jnp.float32),
    ],
)
def _k2_inv(c0_hbm, c1_hbm, inv_hbm, a_v, b_v, o_v):
    c = lax.axis_index("c")
    s = lax.axis_index("s")
    w = s * _NC + c
    base = w * _KSL
    pltpu.sync_copy(c0_hbm.at[pl.ds(base, _KSL)], a_v)
    pltpu.sync_copy(c1_hbm.at[pl.ds(base, _KSL)], b_v)

    @pl.loop(0, _KSL // 16)
    def _(i):
        sl = pl.ds(i * 16, 16)
        o_v[sl] = 1.0 / (a_v[sl] + b_v[sl])

    pltpu.sync_copy(o_v, inv_hbm.at[pl.ds(base, _KSL)])


@functools.partial(
    pl.kernel,
    out_type=[
        jax.ShapeDtypeStruct((_E,), jnp.float32),
        jax.ShapeDtypeStruct((_N, _EMB), jnp.float32),
        jax.ShapeDtypeStruct((_N, _EMB), jnp.float32),
        jax.ShapeDtypeStruct((_BINS,), jnp.float32),
        jax.ShapeDtypeStruct((_BINS,), jnp.float32),
    ],
    mesh=_mesh,
    compiler_params=_sc_params,
    scratch_types=[
        pltpu.VMEM((_EPW,), jnp.int32),
        pltpu.VMEM((_EPW,), jnp.int32),
        pltpu.VMEM((_EPW,), jnp.int32),
        pltpu.VMEM((_EPW,), jnp.float32),
        pltpu.VMEM((_EPW,), jnp.float32),
        pltpu.VMEM((_NCH, _CH), jnp.int32),
        pltpu.VMEM((_NCH, _CH), jnp.int32),
        pltpu.VMEM((_NCH, _CH), jnp.int32),
        pltpu.VMEM((2, _CH, _EMB), jnp.float32),
        pltpu.VMEM((2, _CH, _EMB), jnp.float32),
        pltpu.VMEM((2, _CH), jnp.float32),
        pltpu.VMEM((2, _CH), jnp.float32),
        pltpu.VMEM_SHARED((_N, _EMB), jnp.float32),
        pltpu.VMEM_SHARED((_BINS,), jnp.float32),
        pltpu.SemaphoreType.DMA,
        pltpu.SemaphoreType.DMA,
        pltpu.SemaphoreType.DMA,
        pltpu.SemaphoreType.DMA,
    ],
)
def _k3_edges(src_hbm, dst_hbm, et_hbm, mask_hbm, inv_hbm, w1_hbm,
              zeros_hbm, zerosh_hbm,
              mvals_hbm, h0_hbm, h1_hbm, w0_hbm, w1p_hbm,
              src_l, dst_l, et_l, mask_l, mvals_l,
              ver_v, hor_v, src_v,
              grows, srows, ginv, swval,
              h_sh, w_sh, gsem0, gsem1, ssem0, ssem1):
    c = lax.axis_index("c")
    s = lax.axis_index("s")
    w = s * _NC + c
    nch = jnp.where(w == _NW - 1, _TAIL, _NCH)
    gsem = (gsem0, gsem1)
    ssem = (ssem0, ssem1)
    @pl.when(s < 15)
    def _():
        pltpu.sync_copy(zeros_hbm.at[pl.ds(s * _WSL, _WSL)],
                        w_sh.at[pl.ds(s * _WSL, _WSL)])

    @pl.when(s == 15)
    def _():
        pltpu.sync_copy(zeros_hbm.at[pl.ds(15 * _WSL, _WSL15)],
                        w_sh.at[pl.ds(15 * _WSL, _WSL15)])

    pltpu.sync_copy(zerosh_hbm.at[pl.ds(s * _HROWS, _HROWS)],
                    h_sh.at[pl.ds(s * _HROWS, _HROWS)])
    _load_worker_edges(src_hbm, src_l, w)
    _load_worker_edges(dst_hbm, dst_l, w)
    _load_worker_edges(et_hbm, et_l, w)
    _load_worker_edges(mask_hbm, mask_l, w)

    @pl.loop(0, nch)
    def _(j):
        for k in range(_CH // 16):
            sl1 = pl.ds(j * _CH + k * 16, 16)
            sl = pl.ds(k * 16, 16)
            et16 = et_l[sl1] * _N
            src16 = src_l[sl1]
            ver_v[j, sl] = et16 + src16
            hor_v[j, sl] = et16 + dst_l[sl1]
            src_v[j, sl] = src16

    plsc.subcore_barrier()

    for b in range(2):
        pltpu.async_copy(w1_hbm.at[hor_v.at[b]], grows.at[b], gsem[b])
        pltpu.async_copy(inv_hbm.at[ver_v.at[b]], ginv.at[b], gsem[b])

    @pl.loop(0, nch, step=2)
    def _(jj):
        for b in range(2):
            j = jj + b
            pltpu.make_async_copy(w1_hbm.at[hor_v.at[j]],
                                  grows.at[b], gsem[b]).wait()
            pltpu.make_async_copy(inv_hbm.at[ver_v.at[j]],
                                  ginv.at[b], gsem[b]).wait()

            @pl.when(j >= 2)
            def _():
                pltpu.make_async_copy(srows.at[b], h_sh.at[src_v.at[j]],
                                      ssem[b]).wait()
                pltpu.make_async_copy(swval.at[b], w_sh.at[hor_v.at[j]],
                                      ssem[b]).wait()

            for k in range(_CH // 16):
                sl1 = pl.ds(j * _CH + k * 16, 16)
                sl = pl.ds(k * 16, 16)
                m16 = mask_l[sl1]
                s16 = 1.0 / (1.0 + jnp.exp(-m16))
                mv16 = s16 * ginv[b, sl]
                mvals_l[sl1] = mv16
                src16 = src_l[sl1]
                swval[b, sl] = jnp.where(src16 == _NODE, mv16, 0.0)
                for t in range(16):
                    e = k * 16 + t
                    srows[b, e, :] = grows[b, e, :] * _splat_lane(mv16, t)

            pltpu.async_copy(srows.at[b], h_sh.at[src_v.at[j]],
                             ssem[b], add=True)
            pltpu.async_copy(swval.at[b], w_sh.at[hor_v.at[j]],
                             ssem[b], add=True)

            @pl.when(j + 2 < nch)
            def _():
                pltpu.async_copy(w1_hbm.at[hor_v.at[j + 2]],
                                 grows.at[b], gsem[b])
                pltpu.async_copy(inv_hbm.at[ver_v.at[j + 2]],
                                 ginv.at[b], gsem[b])

    for b in range(2):
        pltpu.make_async_copy(srows.at[b], h_sh.at[src_v.at[0]],
                              ssem[b]).wait()
        pltpu.make_async_copy(swval.at[b], w_sh.at[hor_v.at[0]],
                              ssem[b]).wait()

    @pl.when(w < _NW - 1)
    def _():
        pltpu.sync_copy(mvals_l, mvals_hbm.at[pl.ds(w * _EPW, _EPW)])

    @pl.when(w == _NW - 1)
    def _():
        pltpu.sync_copy(mvals_l.at[pl.ds(0, _TAILE)],
                        mvals_hbm.at[pl.ds((_NW - 1) * _EPW, _TAILE)])

    plsc.subcore_barrier()

    @pl.when(c == 0)
    def _():
        pltpu.sync_copy(h_sh.at[pl.ds(s * _HROWS, _HROWS)],
                        h0_hbm.at[pl.ds(s * _HROWS, _HROWS)])

        @pl.when(s < 15)
        def _():
            pltpu.sync_copy(w_sh.at[pl.ds(s * _WSL, _WSL)],
                            w0_hbm.at[pl.ds(s * _WSL, _WSL)])

        @pl.when(s == 15)
        def _():
            pltpu.sync_copy(w_sh.at[pl.ds(15 * _WSL, _WSL15)],
                            w0_hbm.at[pl.ds(15 * _WSL, _WSL15)])

    @pl.when(c == 1)
    def _():
        pltpu.sync_copy(h_sh.at[pl.ds(s * _HROWS, _HROWS)],
                        h1_hbm.at[pl.ds(s * _HROWS, _HROWS)])

        @pl.when(s < 15)
        def _():
            pltpu.sync_copy(w_sh.at[pl.ds(s * _WSL, _WSL)],
                            w1p_hbm.at[pl.ds(s * _WSL, _WSL)])

        @pl.when(s == 15)
        def _():
            pltpu.sync_copy(w_sh.at[pl.ds(15 * _WSL, _WSL15)],
                            w1p_hbm.at[pl.ds(15 * _WSL, _WSL15)])


def _k4_body(h0_ref, h1_ref, w0_ref, w1_ref, w2t_ref, b1_ref, b2_ref, o_ref):
    h = jnp.maximum(h0_ref[...] + h1_ref[...] + b1_ref[...], 0.0)
    wmat = w0_ref[...] + w1_ref[...]
    t = jnp.dot(wmat, h, preferred_element_type=jnp.float32)
    y = b2_ref[...]
    iota = lax.broadcasted_iota(jnp.int32, (1, 4), 1)
    for o in range(4):
        yo = jnp.sum(t * w2t_ref[o])
        y = y + jnp.where(iota == o, yo, 0.0)
    m = jnp.max(y)
    e = jnp.exp(y - m)
    o_ref[...] = e / jnp.sum(e)


def kernel(edge_mask, weights1, bias1, weights2, bias2, edge_index, edge_type):
    src = edge_index[0].astype(jnp.int32)
    dst = edge_index[1].astype(jnp.int32)
    et = edge_type.astype(jnp.int32)
    zeros1 = jnp.zeros((_CBINS,), jnp.float32)
    zerosh = jnp.zeros((_N, _EMB), jnp.float32)
    w1t = weights1.reshape(_BINS, _EMB)

    c0, c1 = _k1_counts(src, et, zeros1)
    inv = _k2_inv(c0, c1)

    mvals, h0, h1, w0, w1p = _k3_edges(src, dst, et, edge_mask, inv, w1t,
                                       zeros1, zerosh)

    wm0 = w0.reshape(_R, _N)
    wm1 = w1p.reshape(_R, _N)
    w2t = jnp.transpose(weights2, (2, 0, 1))
    res = pl.pallas_call(
        _k4_body,
        out_shape=jax.ShapeDtypeStruct((1, 4), jnp.float32),
    )(h0, h1, wm0, wm1, w2t,
      bias1.reshape(1, _EMB), bias2.reshape(1, 4))

    return (res.reshape(4), mvals)


# speedup vs baseline: 1.1670x; 1.1670x over previous
import functools

import jax
import jax.numpy as jnp
from jax import lax
from jax.experimental import pallas as pl
from jax.experimental.pallas import tpu as pltpu
from jax.experimental.pallas import tpu_sc as plsc

_N = 10000
_R = 17
_E = 320000
_EMB = 16
_NODE = 123
_BINS = _R * _N
_WSL = 10624
_WSL15 = _BINS - 15 * _WSL
_CBINS = 180224
_CSL = _CBINS // 16
_HROWS = _N // 16
_NC = 2
_NS = 16
_NW = _NC * _NS
_CH = 128
_NCH = 80
_NROWS = _E // _CH
_TAIL = _NROWS - (_NW - 1) * _NCH

_mesh = plsc.VectorSubcoreMesh(core_axis_name="c", subcore_axis_name="s")
_sc_params = pltpu.CompilerParams(use_tc_tiling_on_sc=False)

_GATHER_DNUMS = lax.GatherDimensionNumbers(
    offset_dims=(), collapsed_slice_dims=(0,), start_index_map=(0,))


def _splat_lane(vec16, t):
    idx = jnp.full((16, 1), t, jnp.int32)
    return lax.gather(vec16, idx, _GATHER_DNUMS, (1,),
                      mode=lax.GatherScatterMode.PROMISE_IN_BOUNDS)


def _load_worker_rows(hbm, vmem, w):
    @pl.when(w < _NW - 1)
    def _():
        pltpu.sync_copy(hbm.at[pl.ds(w * _NCH, _NCH)], vmem)

    @pl.when(w == _NW - 1)
    def _():
        pltpu.sync_copy(hbm.at[pl.ds((_NW - 1) * _NCH, _TAIL)],
                        vmem.at[pl.ds(0, _TAIL)])


@functools.partial(
    pl.kernel,
    out_type=[
        jax.ShapeDtypeStruct((_CBINS,), jnp.float32),
        jax.ShapeDtypeStruct((_CBINS,), jnp.float32),
    ],
    mesh=_mesh,
    compiler_params=_sc_params,
    scratch_types=[
        pltpu.VMEM((_NCH, _CH), jnp.int32),
        pltpu.VMEM((_CH,), jnp.float32),
        pltpu.VMEM_SHARED((_CBINS,), jnp.float32),
        pltpu.SemaphoreType.DMA,
    ],
)
def _k1_counts(ver_hbm, zeros_hbm, c0_hbm, c1_hbm, ver_v, ones_v, counts_sh,
               sem):
    c = lax.axis_index("c")
    s = lax.axis_index("s")
    w = s * _NC + c
    nch = jnp.where(w == _NW - 1, _TAIL, _NCH)
    pltpu.sync_copy(zeros_hbm.at[pl.ds(s * _CSL, _CSL)],
                    counts_sh.at[pl.ds(s * _CSL, _CSL)])
    for k in range(_CH // 16):
        ones_v[pl.ds(k * 16, 16)] = jnp.full((16,), 1.0, jnp.float32)
    _load_worker_rows(ver_hbm, ver_v, w)
    plsc.subcore_barrier()

    @pl.loop(0, nch)
    def _(j):
        pltpu.async_copy(ones_v, counts_sh.at[ver_v.at[j]], sem, add=True)

    @pl.loop(0, nch)
    def _(j):
        pltpu.make_async_copy(ones_v, counts_sh.at[ver_v.at[j]], sem).wait()

    plsc.subcore_barrier()

    @pl.when(c == 0)
    def _():
        pltpu.sync_copy(counts_sh.at[pl.ds(s * _CSL, _CSL)],
                        c0_hbm.at[pl.ds(s * _CSL, _CSL)])

    @pl.when(c == 1)
    def _():
        pltpu.sync_copy(counts_sh.at[pl.ds(s * _CSL, _CSL)],
                        c1_hbm.at[pl.ds(s * _CSL, _CSL)])


def _k2_body(a_ref, b_ref, o_ref):
    o_ref[...] = 1.0 / (a_ref[...] + b_ref[...])


@functools.partial(
    pl.kernel,
    out_type=[
        jax.ShapeDtypeStruct((_NROWS, _CH), jnp.float32),
        jax.ShapeDtypeStruct((_N, _EMB), jnp.float32),
        jax.ShapeDtypeStruct((_N, _EMB), jnp.float32),
        jax.ShapeDtypeStruct((_BINS,), jnp.float32),
        jax.ShapeDtypeStruct((_BINS,), jnp.float32),
    ],
    mesh=_mesh,
    compiler_params=_sc_params,
    scratch_types=[
        pltpu.VMEM((_NCH, _CH), jnp.int32),
        pltpu.VMEM((_NCH, _CH), jnp.int32),
        pltpu.VMEM((_NCH, _CH), jnp.int32),
        pltpu.VMEM((_NCH, _CH), jnp.float32),
        pltpu.VMEM((_NCH, _CH), jnp.float32),
        pltpu.VMEM((2, _CH, _EMB), jnp.float32),
        pltpu.VMEM((2, _CH, _EMB), jnp.float32),
        pltpu.VMEM((2, _CH), jnp.float32),
        pltpu.VMEM((2, _CH), jnp.float32),
        pltpu.VMEM_SHARED((_N, _EMB), jnp.float32),
        pltpu.VMEM_SHARED((_BINS,), jnp.float32),
        pltpu.SemaphoreType.DMA,
        pltpu.SemaphoreType.DMA,
        pltpu.SemaphoreType.DMA,
        pltpu.SemaphoreType.DMA,
    ],
)
def _k3_edges(ver_hbm, hor_hbm, src_hbm, mask_hbm, inv_hbm, w1_hbm,
              zeros_hbm, zerosh_hbm,
              mvals_hbm, h0_hbm, h1_hbm, w0_hbm, w1p_hbm,
              ver_v, hor_v, src_v, mask_v, mvals_v,
              grows, srows, ginv, swval,
              h_sh, w_sh, gsem0, gsem1, ssem0, ssem1):
    c = lax.axis_index("c")
    s = lax.axis_index("s")
    w = s * _NC + c
    nch = jnp.where(w == _NW - 1, _TAIL, _NCH)
    gsem = (gsem0, gsem1)
    ssem = (ssem0, ssem1)
    @pl.when(s < 15)
    def _():
        pltpu.sync_copy(zeros_hbm.at[pl.ds(s * _WSL, _WSL)],
                        w_sh.at[pl.ds(s * _WSL, _WSL)])

    @pl.when(s == 15)
    def _():
        pltpu.sync_copy(zeros_hbm.at[pl.ds(15 * _WSL, _WSL15)],
                        w_sh.at[pl.ds(15 * _WSL, _WSL15)])

    pltpu.sync_copy(zerosh_hbm.at[pl.ds(s * _HROWS, _HROWS)],
                    h_sh.at[pl.ds(s * _HROWS, _HROWS)])
    _load_worker_rows(ver_hbm, ver_v, w)
    _load_worker_rows(hor_hbm, hor_v, w)
    _load_worker_rows(src_hbm, src_v, w)
    _load_worker_rows(mask_hbm, mask_v, w)
    plsc.subcore_barrier()

    for b in range(2):
        pltpu.async_copy(w1_hbm.at[hor_v.at[b]], grows.at[b], gsem[b])
        pltpu.async_copy(inv_hbm.at[ver_v.at[b]], ginv.at[b], gsem[b])

    @pl.loop(0, nch, step=2)
    def _(jj):
        for b in range(2):
            j = jj + b
            pltpu.make_async_copy(w1_hbm.at[hor_v.at[j]],
                                  grows.at[b], gsem[b]).wait()
            pltpu.make_async_copy(inv_hbm.at[ver_v.at[j]],
                                  ginv.at[b], gsem[b]).wait()

            @pl.when(j >= 2)
            def _():
                pltpu.make_async_copy(srows.at[b], h_sh.at[src_v.at[j]],
                                      ssem[b]).wait()
                pltpu.make_async_copy(swval.at[b], w_sh.at[hor_v.at[j]],
                                      ssem[b]).wait()

            for k in range(_CH // 16):
                sl = pl.ds(k * 16, 16)
                m16 = mask_v[j, sl]
                s16 = 1.0 / (1.0 + jnp.exp(-m16))
                mv16 = s16 * ginv[b, sl]
                mvals_v[j, sl] = mv16
                src16 = src_v[j, sl]
                swval[b, sl] = jnp.where(src16 == _NODE, mv16, 0.0)
                for t in range(16):
                    e = k * 16 + t
                    srows[b, e, :] = grows[b, e, :] * _splat_lane(mv16, t)

            pltpu.async_copy(srows.at[b], h_sh.at[src_v.at[j]],
                             ssem[b], add=True)
            pltpu.async_copy(swval.at[b], w_sh.at[hor_v.at[j]],
                             ssem[b], add=True)

            @pl.when(j + 2 < nch)
            def _():
                pltpu.async_copy(w1_hbm.at[hor_v.at[j + 2]],
                                 grows.at[b], gsem[b])
                pltpu.async_copy(inv_hbm.at[ver_v.at[j + 2]],
                                 ginv.at[b], gsem[b])

    for b in range(2):
        pltpu.make_async_copy(srows.at[b], h_sh.at[src_v.at[0]],
                              ssem[b]).wait()
        pltpu.make_async_copy(swval.at[b], w_sh.at[hor_v.at[0]],
                              ssem[b]).wait()

    @pl.when(w < _NW - 1)
    def _():
        pltpu.sync_copy(mvals_v, mvals_hbm.at[pl.ds(w * _NCH, _NCH)])

    @pl.when(w == _NW - 1)
    def _():
        pltpu.sync_copy(mvals_v.at[pl.ds(0, _TAIL)],
                        mvals_hbm.at[pl.ds((_NW - 1) * _NCH, _TAIL)])

    plsc.subcore_barrier()

    @pl.when(c == 0)
    def _():
        pltpu.sync_copy(h_sh.at[pl.ds(s * _HROWS, _HROWS)],
                        h0_hbm.at[pl.ds(s * _HROWS, _HROWS)])

        @pl.when(s < 15)
        def _():
            pltpu.sync_copy(w_sh.at[pl.ds(s * _WSL, _WSL)],
                            w0_hbm.at[pl.ds(s * _WSL, _WSL)])

        @pl.when(s == 15)
        def _():
            pltpu.sync_copy(w_sh.at[pl.ds(15 * _WSL, _WSL15)],
                            w0_hbm.at[pl.ds(15 * _WSL, _WSL15)])

    @pl.when(c == 1)
    def _():
        pltpu.sync_copy(h_sh.at[pl.ds(s * _HROWS, _HROWS)],
                        h1_hbm.at[pl.ds(s * _HROWS, _HROWS)])

        @pl.when(s < 15)
        def _():
            pltpu.sync_copy(w_sh.at[pl.ds(s * _WSL, _WSL)],
                            w1p_hbm.at[pl.ds(s * _WSL, _WSL)])

        @pl.when(s == 15)
        def _():
            pltpu.sync_copy(w_sh.at[pl.ds(15 * _WSL, _WSL15)],
                            w1p_hbm.at[pl.ds(15 * _WSL, _WSL15)])


def _k4_body(h0_ref, h1_ref, w0_ref, w1_ref, w2t_ref, b1_ref, b2_ref, o_ref):
    h = jnp.maximum(h0_ref[...] + h1_ref[...] + b1_ref[...], 0.0)
    wmat = w0_ref[...] + w1_ref[...]
    t = jnp.dot(wmat, h, preferred_element_type=jnp.float32)
    y = b2_ref[...]
    iota = lax.broadcasted_iota(jnp.int32, (1, 4), 1)
    for o in range(4):
        yo = jnp.sum(t * w2t_ref[o])
        y = y + jnp.where(iota == o, yo, 0.0)
    m = jnp.max(y)
    e = jnp.exp(y - m)
    o_ref[...] = e / jnp.sum(e)


def kernel(edge_mask, weights1, bias1, weights2, bias2, edge_index, edge_type):
    src = edge_index[0].astype(jnp.int32)
    dst = edge_index[1].astype(jnp.int32)
    et = edge_type.astype(jnp.int32)
    ver = (et * _N + src).reshape(_NROWS, _CH)
    hor = (et * _N + dst).reshape(_NROWS, _CH)
    src2 = src.reshape(_NROWS, _CH)
    mask2 = edge_mask.reshape(_NROWS, _CH)
    zeros1 = jnp.zeros((_CBINS,), jnp.float32)
    zerosh = jnp.zeros((_N, _EMB), jnp.float32)
    w1t = weights1.reshape(_BINS, _EMB)

    c0, c1 = _k1_counts(ver, zeros1)
    inv = pl.pallas_call(
        _k2_body,
        out_shape=jax.ShapeDtypeStruct((_CBINS // 128, 128), jnp.float32),
    )(c0.reshape(_CBINS // 128, 128),
      c1.reshape(_CBINS // 128, 128)).reshape(_CBINS)

    mvals2, h0, h1, w0, w1p = _k3_edges(ver, hor, src2, mask2, inv, w1t,
                                        zeros1, zerosh)

    wm0 = w0.reshape(_R, _N)
    wm1 = w1p.reshape(_R, _N)
    w2t = jnp.transpose(weights2, (2, 0, 1))
    res = pl.pallas_call(
        _k4_body,
        out_shape=jax.ShapeDtypeStruct((1, 4), jnp.float32),
    )(h0, h1, wm0, wm1, w2t,
      bias1.reshape(1, _EMB), bias2.reshape(1, 4))

    return (res.reshape(4), mvals2.reshape(_E))


# speedup vs baseline: 1.3175x; 1.1290x over previous
import functools

import jax
import jax.numpy as jnp
from jax import lax
from jax.experimental import pallas as pl
from jax.experimental.pallas import tpu as pltpu
from jax.experimental.pallas import tpu_sc as plsc

_N = 10000
_R = 17
_E = 320000
_EMB = 16
_NODE = 123
_BINS = _R * _N
_WSL = 10624
_WSL15 = _BINS - 15 * _WSL
_CBINS = 180224
_CSL = _CBINS // 16
_HROWS = _N // 16
_NC = 2
_NS = 16
_NW = _NC * _NS
_CH = 128
_NCH = 80
_NROWS = _E // _CH
_TAIL = _NROWS - (_NW - 1) * _NCH

_mesh = plsc.VectorSubcoreMesh(core_axis_name="c", subcore_axis_name="s")
_sc_params = pltpu.CompilerParams(use_tc_tiling_on_sc=False)

_GATHER_DNUMS = lax.GatherDimensionNumbers(
    offset_dims=(), collapsed_slice_dims=(0,), start_index_map=(0,))


def _splat_lane(vec16, t):
    idx = jnp.full((16, 1), t, jnp.int32)
    return lax.gather(vec16, idx, _GATHER_DNUMS, (1,),
                      mode=lax.GatherScatterMode.PROMISE_IN_BOUNDS)


def _load_worker_rows(hbm, vmem, w):
    @pl.when(w < _NW - 1)
    def _():
        pltpu.sync_copy(hbm.at[pl.ds(w * _NCH, _NCH)], vmem)

    @pl.when(w == _NW - 1)
    def _():
        pltpu.sync_copy(hbm.at[pl.ds((_NW - 1) * _NCH, _TAIL)],
                        vmem.at[pl.ds(0, _TAIL)])


@functools.partial(
    pl.kernel,
    out_type=[
        jax.ShapeDtypeStruct((_CBINS,), jnp.float32),
        jax.ShapeDtypeStruct((_CBINS,), jnp.float32),
    ],
    mesh=_mesh,
    compiler_params=_sc_params,
    scratch_types=[
        pltpu.VMEM((_NCH, _CH), jnp.int32),
        pltpu.VMEM((_CH,), jnp.float32),
        pltpu.VMEM_SHARED((_CBINS,), jnp.float32),
        pltpu.SemaphoreType.DMA,
    ],
)
def _k1_counts(ver_hbm, zeros_hbm, c0_hbm, c1_hbm, ver_v, ones_v, counts_sh,
               sem):
    c = lax.axis_index("c")
    s = lax.axis_index("s")
    w = s * _NC + c
    nch = jnp.where(w == _NW - 1, _TAIL, _NCH)
    pltpu.sync_copy(zeros_hbm.at[pl.ds(s * _CSL, _CSL)],
                    counts_sh.at[pl.ds(s * _CSL, _CSL)])
    for k in range(_CH // 16):
        ones_v[pl.ds(k * 16, 16)] = jnp.full((16,), 1.0, jnp.float32)
    _load_worker_rows(ver_hbm, ver_v, w)
    plsc.subcore_barrier()

    @pl.loop(0, nch)
    def _(j):
        pltpu.async_copy(ones_v, counts_sh.at[ver_v.at[j]], sem, add=True)

    @pl.loop(0, nch)
    def _(j):
        pltpu.make_async_copy(ones_v, counts_sh.at[ver_v.at[j]], sem).wait()

    plsc.subcore_barrier()

    @pl.when(c == 0)
    def _():
        pltpu.sync_copy(counts_sh.at[pl.ds(s * _CSL, _CSL)],
                        c0_hbm.at[pl.ds(s * _CSL, _CSL)])

    @pl.when(c == 1)
    def _():
        pltpu.sync_copy(counts_sh.at[pl.ds(s * _CSL, _CSL)],
                        c1_hbm.at[pl.ds(s * _CSL, _CSL)])


def _k2_body(a_ref, b_ref, o_ref):
    o_ref[...] = 1.0 / (a_ref[...] + b_ref[...])


@functools.partial(
    pl.kernel,
    out_type=[
        jax.ShapeDtypeStruct((_NROWS, _CH), jnp.float32),
        jax.ShapeDtypeStruct((_N, _EMB), jnp.float32),
        jax.ShapeDtypeStruct((_N, _EMB), jnp.float32),
        jax.ShapeDtypeStruct((_BINS,), jnp.float32),
        jax.ShapeDtypeStruct((_BINS,), jnp.float32),
    ],
    mesh=_mesh,
    compiler_params=_sc_params,
    scratch_types=[
        pltpu.VMEM((_NCH, _CH), jnp.int32),
        pltpu.VMEM((_NCH, _CH), jnp.int32),
        pltpu.VMEM((_NCH, _CH), jnp.int32),
        pltpu.VMEM((_NCH, _CH), jnp.float32),
        pltpu.VMEM((_NCH, _CH), jnp.float32),
        pltpu.VMEM((4, _CH, _EMB), jnp.float32),
        pltpu.VMEM((4, _CH, _EMB), jnp.float32),
        pltpu.VMEM((4, _CH), jnp.float32),
        pltpu.VMEM((4, _CH), jnp.float32),
        pltpu.VMEM_SHARED((_N, _EMB), jnp.float32),
        pltpu.VMEM_SHARED((_BINS,), jnp.float32),
        pltpu.SemaphoreType.DMA,
        pltpu.SemaphoreType.DMA,
        pltpu.SemaphoreType.DMA,
        pltpu.SemaphoreType.DMA,
        pltpu.SemaphoreType.DMA,
        pltpu.SemaphoreType.DMA,
        pltpu.SemaphoreType.DMA,
        pltpu.SemaphoreType.DMA,
    ],
)
def _k3_edges(ver_hbm, hor_hbm, src_hbm, mask_hbm, inv_hbm, w1_hbm,
              zeros_hbm, zerosh_hbm,
              mvals_hbm, h0_hbm, h1_hbm, w0_hbm, w1p_hbm,
              ver_v, hor_v, src_v, mask_v, mvals_v,
              grows, srows, ginv, swval,
              h_sh, w_sh, gsem0, gsem1, gsem2, gsem3,
              ssem0, ssem1, ssem2, ssem3):
    c = lax.axis_index("c")
    s = lax.axis_index("s")
    w = s * _NC + c
    nch = jnp.where(w == _NW - 1, _TAIL, _NCH)
    gsem = (gsem0, gsem1, gsem2, gsem3)
    ssem = (ssem0, ssem1, ssem2, ssem3)
    @pl.when(s < 15)
    def _():
        pltpu.sync_copy(zeros_hbm.at[pl.ds(s * _WSL, _WSL)],
                        w_sh.at[pl.ds(s * _WSL, _WSL)])

    @pl.when(s == 15)
    def _():
        pltpu.sync_copy(zeros_hbm.at[pl.ds(15 * _WSL, _WSL15)],
                        w_sh.at[pl.ds(15 * _WSL, _WSL15)])

    pltpu.sync_copy(zerosh_hbm.at[pl.ds(s * _HROWS, _HROWS)],
                    h_sh.at[pl.ds(s * _HROWS, _HROWS)])
    _load_worker_rows(ver_hbm, ver_v, w)
    _load_worker_rows(hor_hbm, hor_v, w)
    _load_worker_rows(src_hbm, src_v, w)
    _load_worker_rows(mask_hbm, mask_v, w)
    plsc.subcore_barrier()

    for b in range(4):
        pltpu.async_copy(w1_hbm.at[hor_v.at[b]], grows.at[b], gsem[b])
        pltpu.async_copy(inv_hbm.at[ver_v.at[b]], ginv.at[b], gsem[b])

    @pl.loop(0, nch, step=4)
    def _(jj):
        for b in range(4):
            j = jj + b
            pltpu.make_async_copy(w1_hbm.at[hor_v.at[j]],
                                  grows.at[b], gsem[b]).wait()
            pltpu.make_async_copy(inv_hbm.at[ver_v.at[j]],
                                  ginv.at[b], gsem[b]).wait()

            @pl.when(j >= 4)
            def _():
                pltpu.make_async_copy(srows.at[b], h_sh.at[src_v.at[j]],
                                      ssem[b]).wait()
                pltpu.make_async_copy(swval.at[b], w_sh.at[hor_v.at[j]],
                                      ssem[b]).wait()

            for k in range(_CH // 16):
                sl = pl.ds(k * 16, 16)
                m16 = mask_v[j, sl]
                s16 = 1.0 / (1.0 + jnp.exp(-m16))
                mv16 = s16 * ginv[b, sl]
                mvals_v[j, sl] = mv16
                src16 = src_v[j, sl]
                swval[b, sl] = jnp.where(src16 == _NODE, mv16, 0.0)
                for t in range(16):
                    e = k * 16 + t
                    srows[b, e, :] = grows[b, e, :] * _splat_lane(mv16, t)

            pltpu.async_copy(srows.at[b], h_sh.at[src_v.at[j]],
                             ssem[b], add=True)
            pltpu.async_copy(swval.at[b], w_sh.at[hor_v.at[j]],
                             ssem[b], add=True)

            @pl.when(j + 4 < nch)
            def _():
                pltpu.async_copy(w1_hbm.at[hor_v.at[j + 4]],
                                 grows.at[b], gsem[b])
                pltpu.async_copy(inv_hbm.at[ver_v.at[j + 4]],
                                 ginv.at[b], gsem[b])

    for b in range(4):
        pltpu.make_async_copy(srows.at[b], h_sh.at[src_v.at[0]],
                              ssem[b]).wait()
        pltpu.make_async_copy(swval.at[b], w_sh.at[hor_v.at[0]],
                              ssem[b]).wait()

    @pl.when(w < _NW - 1)
    def _():
        pltpu.sync_copy(mvals_v, mvals_hbm.at[pl.ds(w * _NCH, _NCH)])

    @pl.when(w == _NW - 1)
    def _():
        pltpu.sync_copy(mvals_v.at[pl.ds(0, _TAIL)],
                        mvals_hbm.at[pl.ds((_NW - 1) * _NCH, _TAIL)])

    plsc.subcore_barrier()

    @pl.when(c == 0)
    def _():
        pltpu.sync_copy(h_sh.at[pl.ds(s * _HROWS, _HROWS)],
                        h0_hbm.at[pl.ds(s * _HROWS, _HROWS)])

        @pl.when(s < 15)
        def _():
            pltpu.sync_copy(w_sh.at[pl.ds(s * _WSL, _WSL)],
                            w0_hbm.at[pl.ds(s * _WSL, _WSL)])

        @pl.when(s == 15)
        def _():
            pltpu.sync_copy(w_sh.at[pl.ds(15 * _WSL, _WSL15)],
                            w0_hbm.at[pl.ds(15 * _WSL, _WSL15)])

    @pl.when(c == 1)
    def _():
        pltpu.sync_copy(h_sh.at[pl.ds(s * _HROWS, _HROWS)],
                        h1_hbm.at[pl.ds(s * _HROWS, _HROWS)])

        @pl.when(s < 15)
        def _():
            pltpu.sync_copy(w_sh.at[pl.ds(s * _WSL, _WSL)],
                            w1p_hbm.at[pl.ds(s * _WSL, _WSL)])

        @pl.when(s == 15)
        def _():
            pltpu.sync_copy(w_sh.at[pl.ds(15 * _WSL, _WSL15)],
                            w1p_hbm.at[pl.ds(15 * _WSL, _WSL15)])


def _k4_body(h0_ref, h1_ref, w0_ref, w1_ref, w2t_ref, b1_ref, b2_ref, o_ref):
    h = jnp.maximum(h0_ref[...] + h1_ref[...] + b1_ref[...], 0.0)
    wmat = w0_ref[...] + w1_ref[...]
    t = jnp.dot(wmat, h, preferred_element_type=jnp.float32)
    y = b2_ref[...]
    iota = lax.broadcasted_iota(jnp.int32, (1, 4), 1)
    for o in range(4):
        yo = jnp.sum(t * w2t_ref[o])
        y = y + jnp.where(iota == o, yo, 0.0)
    m = jnp.max(y)
    e = jnp.exp(y - m)
    o_ref[...] = e / jnp.sum(e)


def kernel(edge_mask, weights1, bias1, weights2, bias2, edge_index, edge_type):
    src = edge_index[0].astype(jnp.int32)
    dst = edge_index[1].astype(jnp.int32)
    et = edge_type.astype(jnp.int32)
    ver = (et * _N + src).reshape(_NROWS, _CH)
    hor = (et * _N + dst).reshape(_NROWS, _CH)
    src2 = src.reshape(_NROWS, _CH)
    mask2 = edge_mask.reshape(_NROWS, _CH)
    zeros1 = jnp.zeros((_CBINS,), jnp.float32)
    zerosh = jnp.zeros((_N, _EMB), jnp.float32)
    w1t = weights1.reshape(_BINS, _EMB)

    c0, c1 = _k1_counts(ver, zeros1)
    inv = pl.pallas_call(
        _k2_body,
        out_shape=jax.ShapeDtypeStruct((_CBINS // 128, 128), jnp.float32),
    )(c0.reshape(_CBINS // 128, 128),
      c1.reshape(_CBINS // 128, 128)).reshape(_CBINS)

    mvals2, h0, h1, w0, w1p = _k3_edges(ver, hor, src2, mask2, inv, w1t,
                                        zeros1, zerosh)

    wm0 = w0.reshape(_R, _N)
    wm1 = w1p.reshape(_R, _N)
    w2t = jnp.transpose(weights2, (2, 0, 1))
    res = pl.pallas_call(
        _k4_body,
        out_shape=jax.ShapeDtypeStruct((1, 4), jnp.float32),
    )(h0, h1, wm0, wm1, w2t,
      bias1.reshape(1, _EMB), bias2.reshape(1, 4))

    return (res.reshape(4), mvals2.reshape(_E))
